# baseline (speedup 1.0000x reference)
import functools

import jax
import jax.numpy as jnp
from jax.experimental import pallas as pl

B = 64
V = 100000
VP = 100096
K = 64
EPS = 1e-9


def _row_kernel(logits_ref, temp_ref, topk_ref, topp_ref, minp_ref,
                pos_ref, seed_ref, tok_ref, logp_ref):
    x = logits_ref[...].reshape(1, VP)
    temp = temp_ref[0, 0, 0]
    top_k = topk_ref[0, 0, 0]
    top_p = topp_ref[0, 0, 0]
    min_p = minp_ref[0, 0, 0]
    pos = pos_ref[0, 0, 0]
    seed = seed_ref[0, 0, 0]

    iota = jax.lax.broadcasted_iota(jnp.int32, (1, VP), 1)
    real = iota < V

    x = x / temp
    m = jnp.max(x)
    e = jnp.where(real, jnp.exp(x - m), 0.0)
    z = jnp.sum(e)
    probs = e / z

    logp_ref[...] = jnp.clip(jnp.log(probs),
                             min=jnp.finfo(jnp.float32).min
                             ).reshape(1, 1, VP)

    iota64 = jax.lax.broadcasted_iota(jnp.int32, (1, K), 1)

    def body(t, carry):
        cur, full_s, idx64, cum, v0, sumexp = carry
        v = jnp.max(cur)
        sel = jnp.max(jnp.where(cur == v, iota, -1))
        cur = jnp.where(iota == sel, -1.0, cur)
        v0 = jnp.where(t == 0, v, v0)
        cum = cum + v
        s = jnp.where(t >= top_k, 0.0, v)
        s = jnp.where(cum - s > top_p, 0.0, s)
        s = jnp.where(s < v0 * min_p, 0.0, s)
        full_s = jnp.where(iota == t, s, full_s)
        idx64 = jnp.where(iota64 == t, sel, idx64)
        sumexp = sumexp + jnp.exp(s - v0)
        return cur, full_s, idx64, cum, v0, sumexp

    cur0 = jnp.where(real, probs, -1.0)
    full_s0 = jnp.zeros((1, VP), jnp.float32)
    idx640 = jnp.zeros((1, K), jnp.int32)
    carry = (cur0, full_s0, idx640,
             jnp.float32(0.0), jnp.float32(0.0), jnp.float32(0.0))
    _, full_s, idx64, _, v0, sumexp = jax.lax.fori_loop(0, K, body, carry)

    zs = sumexp + jnp.float32(V - K) * jnp.exp(-v0)
    inputs = jnp.exp(full_s - v0) / zs

    step_seed = seed * 19349663 ^ pos * 73856093
    hashed = step_seed * 805306457 ^ iota * 479001599
    u = jnp.mod(hashed, 2 ** 24).astype(jnp.float32) / jnp.float32(2 ** 24)
    gumbel = -jnp.log(-jnp.log(u + EPS) + EPS)
    pert = jnp.log(inputs + EPS) + gumbel
    pert = jnp.where(real, pert, -jnp.float32(1e30))
    pmax = jnp.max(pert)
    r = jnp.min(jnp.where(pert == pmax, iota, VP))

    tok_top = jnp.sum(jnp.where(iota64 == r, idx64, 0))

    keys = jnp.where(real, jax.lax.bitcast_convert_type(probs, jnp.int32),
                     -1)

    def bs_val(_, lh):
        lo, hi = lh
        mid = (lo + hi) // 2
        c = jnp.sum(jnp.where(keys > mid, 1, 0))
        le = c <= r
        return jnp.where(le, lo, mid), jnp.where(le, mid, hi)

    _, vstar = jax.lax.fori_loop(
        0, 31, bs_val, (jnp.int32(-1), jnp.int32(2 ** 30)))

    cgt = jnp.sum(jnp.where(keys > vstar, 1, 0))
    t_ord = r - cgt
    tie = keys == vstar

    def bs_idx(_, lh):
        lo, hi = lh
        mid = (lo + hi) // 2
        c = jnp.sum(jnp.where(tie & (iota > mid), 1, 0))
        le = c <= t_ord
        return jnp.where(le, lo, mid), jnp.where(le, mid, hi)

    _, tok_sel = jax.lax.fori_loop(
        0, 18, bs_idx, (jnp.int32(-1), jnp.int32(VP)))

    tok_ref[...] = jnp.where(r < K, tok_top, tok_sel).reshape(1, 1, 1)


@jax.jit
def kernel(logits, temperatures, top_ks, top_ps, min_ps, positions,
           sampling_seeds):
    lp = jnp.pad(logits, ((0, 0), (0, VP - V)),
                 constant_values=-1e30).reshape(B, 1, VP)
    col = lambda a: a.reshape(B, 1, 1)
    row_spec = pl.BlockSpec((1, 1, VP), lambda i: (i, 0, 0))
    s_spec = pl.BlockSpec((1, 1, 1), lambda i: (i, 0, 0))
    tok, logp = pl.pallas_call(
        _row_kernel,
        grid=(B,),
        in_specs=[row_spec] + [s_spec] * 6,
        out_specs=[s_spec, row_spec],
        out_shape=[jax.ShapeDtypeStruct((B, 1, 1), jnp.int32),
                   jax.ShapeDtypeStruct((B, 1, VP), jnp.float32)],
    )(lp, col(temperatures), col(top_ks), col(top_ps), col(min_ps),
      col(positions), col(sampling_seeds))
    return tok.reshape(-1), logp.reshape(B, VP)[:, :V]
